# baseline (speedup 1.0000x reference)
import functools

import jax
import jax.numpy as jnp
from jax import lax
from jax.experimental import pallas as pl
from jax.experimental.pallas import tpu as pltpu
from jax.experimental.pallas import tpu_sc as plsc

B = 32
N = 512
HW = 196
UPDATES = 10
L = 16
CG = 16
NG = N // CG
NC = 2

_mesh = plsc.VectorSubcoreMesh(core_axis_name="c", subcore_axis_name="s")


def _kmeans_body(x_ref, y_ref, m0_ref, m1_ref):
    x = x_ref[...]
    y = y_ref[...]
    cx0 = x[:, 0:1]
    cy0 = y[:, 0:1]
    cx1 = x[:, 1:2]
    cy1 = y[:, 1:2]

    def body(t, c):
        cx0, cy0, cx1, cy1 = c
        d0 = (x - cx0) ** 2 + (y - cy0) ** 2
        d1 = (x - cx1) ** 2 + (y - cy1) ** 2
        m1 = (d1 < d0).astype(jnp.float32)
        m0 = 1.0 - m1
        c0 = jnp.sum(m0, axis=1, keepdims=True)
        c1 = jnp.sum(m1, axis=1, keepdims=True)
        s0x = jnp.sum(x * m0, axis=1, keepdims=True)
        s0y = jnp.sum(y * m0, axis=1, keepdims=True)
        s1x = jnp.sum(x * m1, axis=1, keepdims=True)
        s1y = jnp.sum(y * m1, axis=1, keepdims=True)
        return (s0x / c0, s0y / c0, s1x / c1, s1y / c1)

    cx0, cy0, cx1, cy1 = jax.lax.fori_loop(0, UPDATES, body, (cx0, cy0, cx1, cy1))
    d0 = (x - cx0) ** 2 + (y - cy0) ** 2
    d1 = (x - cx1) ** 2 + (y - cy1) ** 2
    m1 = (d1 < d0).astype(jnp.float32)
    m1_ref[...] = m1
    m0_ref[...] = 1.0 - m1


@functools.partial(
    pl.kernel,
    out_type=[jax.ShapeDtypeStruct((B, N, HW), jnp.float32)] * 2,
    mesh=_mesh,
    compiler_params=pltpu.CompilerParams(needs_layout_passes=False),
    scratch_types=[
        pltpu.VMEM((HW * N,), jnp.float32),
        pltpu.VMEM((2, N), jnp.float32),
        pltpu.VMEM((2, CG, HW), jnp.float32),
        pltpu.VMEM((2, CG, HW), jnp.float32),
        pltpu.SemaphoreType.DMA,
        pltpu.SemaphoreType.DMA,
        pltpu.SemaphoreType.DMA,
        pltpu.SemaphoreType.DMA,
        pltpu.SemaphoreType.DMA,
    ],
)
def _sc_mask_transpose(m0_hbm, m1_hbm, feats_hbm, o0_hbm, o1_hbm,
                       f_v, mk_v, s0_v, s1_v,
                       sem_f, sem00, sem01, sem10, sem11):
    wid = lax.axis_index("s") * NC + lax.axis_index("c")

    fcopy = pltpu.async_copy(feats_hbm.at[wid], f_v, sem_f)
    pltpu.sync_copy(m0_hbm.at[wid], mk_v.at[0])
    pltpu.sync_copy(m1_hbm.at[wid], mk_v.at[1])
    fcopy.wait()

    step_ch = lax.iota(jnp.int32, L)
    onev = jnp.ones((L,), jnp.int32)
    sems0 = (sem00, sem01)
    sems1 = (sem10, sem11)

    for g in range(NG):
        rb = g % 2
        if g >= 2:
            pltpu.make_async_copy(
                s0_v.at[rb], o0_hbm.at[wid, pl.ds((g - 2) * CG, CG)],
                sems0[rb]).wait()
            pltpu.make_async_copy(
                s1_v.at[rb], o1_hbm.at[wid, pl.ds((g - 2) * CG, CG)],
                sems1[rb]).wait()
        base = g * CG
        m0g = mk_v[0, pl.ds(base, L)]
        m1g = mk_v[1, pl.ds(base, L)]

        def j_body(j, jv):
            row = f_v[pl.ds(j * N + base, L)]
            plsc.store_scatter(s0_v.at[rb], [step_ch, jv], row * m0g)
            plsc.store_scatter(s1_v.at[rb], [step_ch, jv], row * m1g)
            return jv + onev

        lax.fori_loop(0, HW, j_body, jnp.zeros((L,), jnp.int32), unroll=7)
        pltpu.async_copy(s0_v.at[rb], o0_hbm.at[wid, pl.ds(base, CG)],
                         sems0[rb])
        pltpu.async_copy(s1_v.at[rb], o1_hbm.at[wid, pl.ds(base, CG)],
                         sems1[rb])

    for g in (NG - 2, NG - 1):
        rb = g % 2
        pltpu.make_async_copy(
            s0_v.at[rb], o0_hbm.at[wid, pl.ds(g * CG, CG)], sems0[rb]).wait()
        pltpu.make_async_copy(
            s1_v.at[rb], o1_hbm.at[wid, pl.ds(g * CG, CG)], sems1[rb]).wait()


def kernel(max_points, feature_batch):
    pts = max_points[:, :, 0, :]
    xs = pts[:, :, 0]
    ys = pts[:, :, 1]
    feats = feature_batch.reshape(B, HW * N)

    m0, m1 = pl.pallas_call(
        _kmeans_body,
        out_shape=[jax.ShapeDtypeStruct((B, N), jnp.float32)] * 2,
    )(xs, ys)

    o0, o1 = _sc_mask_transpose(m0, m1, feats)
    return o0.reshape(B, N, 14, 14), o1.reshape(B, N, 14, 14)


# speedup vs baseline: 1.2151x; 1.2151x over previous
import functools

import jax
import jax.numpy as jnp
from jax import lax
from jax.experimental import pallas as pl
from jax.experimental.pallas import tpu as pltpu
from jax.experimental.pallas import tpu_sc as plsc

B = 32
N = 512
HW = 196
UPDATES = 10
L = 16
CG = 16
NG = N // CG
NC = 2
NP = N

_JSTARTS = [16 * c for c in range(12)] + [HW - L]

_mesh = plsc.VectorSubcoreMesh(core_axis_name="c", subcore_axis_name="s")


def _kmeans_body(x_ref, y_ref, m0_ref, m1_ref):
    x = x_ref[...]
    y = y_ref[...]
    cx0 = x[:, 0:1]
    cy0 = y[:, 0:1]
    cx1 = x[:, 1:2]
    cy1 = y[:, 1:2]

    def body(t, c):
        cx0, cy0, cx1, cy1 = c
        d0 = (x - cx0) ** 2 + (y - cy0) ** 2
        d1 = (x - cx1) ** 2 + (y - cy1) ** 2
        m1 = (d1 < d0).astype(jnp.float32)
        m0 = 1.0 - m1
        c0 = jnp.sum(m0, axis=1, keepdims=True)
        c1 = jnp.sum(m1, axis=1, keepdims=True)
        s0x = jnp.sum(x * m0, axis=1, keepdims=True)
        s0y = jnp.sum(y * m0, axis=1, keepdims=True)
        s1x = jnp.sum(x * m1, axis=1, keepdims=True)
        s1y = jnp.sum(y * m1, axis=1, keepdims=True)
        return (s0x / c0, s0y / c0, s1x / c1, s1y / c1)

    cx0, cy0, cx1, cy1 = jax.lax.fori_loop(0, UPDATES, body, (cx0, cy0, cx1, cy1))
    d0 = (x - cx0) ** 2 + (y - cy0) ** 2
    d1 = (x - cx1) ** 2 + (y - cy1) ** 2
    m1 = (d1 < d0).astype(jnp.float32)
    m1_ref[...] = m1
    m0_ref[...] = 1.0 - m1


@functools.partial(
    pl.kernel,
    out_type=[jax.ShapeDtypeStruct((B, N, HW), jnp.float32)] * 2,
    mesh=_mesh,
    compiler_params=pltpu.CompilerParams(needs_layout_passes=False),
    scratch_types=[
        pltpu.VMEM((HW, NP), jnp.float32),
        pltpu.VMEM((2, N), jnp.float32),
        pltpu.VMEM((2, CG, HW), jnp.float32),
        pltpu.VMEM((2, CG, HW), jnp.float32),
        pltpu.SemaphoreType.DMA,
        pltpu.SemaphoreType.DMA,
        pltpu.SemaphoreType.DMA,
        pltpu.SemaphoreType.DMA,
        pltpu.SemaphoreType.DMA,
    ],
)
def _sc_mask_transpose(m0_hbm, m1_hbm, feats_hbm, o0_hbm, o1_hbm,
                       f_v, mk_v, s0_v, s1_v,
                       sem_f, sem00, sem01, sem10, sem11):
    wid = lax.axis_index("s") * NC + lax.axis_index("c")

    fcopy = pltpu.async_copy(feats_hbm.at[wid], f_v, sem_f)
    pltpu.sync_copy(m0_hbm.at[wid], mk_v.at[0])
    pltpu.sync_copy(m1_hbm.at[wid], mk_v.at[1])
    fcopy.wait()

    iz = jnp.zeros((L,), jnp.int32)
    ione = jnp.full((L,), 1, jnp.int32)
    jvecs = [lax.iota(jnp.int32, L) + j0 for j0 in _JSTARTS]
    sems0 = (sem00, sem01)
    sems1 = (sem10, sem11)

    for g in range(NG):
        rb = g % 2
        if g >= 2:
            pltpu.make_async_copy(
                s0_v.at[rb], o0_hbm.at[wid, pl.ds((g - 2) * CG, CG)],
                sems0[rb]).wait()
            pltpu.make_async_copy(
                s1_v.at[rb], o1_hbm.at[wid, pl.ds((g - 2) * CG, CG)],
                sems1[rb]).wait()
        base = g * CG
        vv0 = jnp.full((L,), base, jnp.int32)

        def ch_body(c, vv):
            m0v = plsc.load_gather(mk_v, [iz, vv])
            m1v = plsc.load_gather(mk_v, [ione, vv])
            for ci, j0 in enumerate(_JSTARTS):
                row = plsc.load_gather(f_v, [jvecs[ci], vv])
                s0_v[rb, c, pl.ds(j0, L)] = row * m0v
                s1_v[rb, c, pl.ds(j0, L)] = row * m1v
            return vv + ione

        lax.fori_loop(0, CG, ch_body, vv0)
        pltpu.async_copy(s0_v.at[rb], o0_hbm.at[wid, pl.ds(base, CG)],
                         sems0[rb])
        pltpu.async_copy(s1_v.at[rb], o1_hbm.at[wid, pl.ds(base, CG)],
                         sems1[rb])

    for g in (NG - 2, NG - 1):
        rb = g % 2
        pltpu.make_async_copy(
            s0_v.at[rb], o0_hbm.at[wid, pl.ds(g * CG, CG)],
            sems0[rb]).wait()
        pltpu.make_async_copy(
            s1_v.at[rb], o1_hbm.at[wid, pl.ds(g * CG, CG)],
            sems1[rb]).wait()


def kernel(max_points, feature_batch):
    pts = max_points[:, :, 0, :]
    xs = pts[:, :, 0]
    ys = pts[:, :, 1]
    feats = feature_batch.reshape(B, HW, N)

    m0, m1 = pl.pallas_call(
        _kmeans_body,
        out_shape=[jax.ShapeDtypeStruct((B, N), jnp.float32)] * 2,
    )(xs, ys)

    o0, o1 = _sc_mask_transpose(m0, m1, feats)
    return o0.reshape(B, N, 14, 14), o1.reshape(B, N, 14, 14)
